# baseline (speedup 1.0000x reference)
import functools

import jax
import jax.numpy as jnp
from jax import lax
from jax.experimental import pallas as pl
from jax.experimental.pallas import tpu as pltpu
from jax.experimental.pallas import tpu_sc as plsc

FEAT = 16
B = 4
HG = 512
WG = 512
N = B * HG * WG
HW = HG * WG

NC = 2
NS = 16
NWORK = NC * NS
SPW = N // NWORK
C = 128
NCHUNK = SPW // C
NL = 4
NJ = 8
NK = 16
XBLK = 8
BW = 1024

LEVELS = ((1024, 0), (512, 1048576), (256, 1310720), (128, 1376256))
VTOT = 1392640


@functools.partial(
    pl.kernel,
    out_type=(
        jax.ShapeDtypeStruct((NC, VTOT, FEAT), jnp.int32),
        jax.ShapeDtypeStruct((B, FEAT, HW), jnp.float32),
    ),
    mesh=plsc.VectorSubcoreMesh(
        core_axis_name="c", subcore_axis_name="s",
        num_cores=NC, num_subcores=NS,
    ),
    compiler_params=pltpu.CompilerParams(
        needs_layout_passes=False, use_tc_tiling_on_sc=False),
    scratch_types=[
        pltpu.VMEM((2, FEAT, BW + 16), jnp.float32),
        pltpu.VMEM((2, BW, FEAT), jnp.int32),
        pltpu.VMEM((XBLK, 2, C), jnp.float32),
        pltpu.VMEM((2, NJ, C), jnp.int32),
        pltpu.VMEM((2, NK, C), jnp.float32),
        pltpu.VMEM((NJ * C, FEAT), jnp.int32),
        pltpu.VMEM((NJ * C, FEAT), jnp.int32),
        pltpu.VMEM((2, FEAT, C), jnp.float32),
        pltpu.SemaphoreType.DMA,
        pltpu.SemaphoreType.DMA,
        pltpu.SemaphoreType.DMA,
        pltpu.SemaphoreType.DMA,
        pltpu.SemaphoreType.DMA,
    ],
)
def _tex_sc_kernel(xq_hbm, t1_hbm, t2_hbm, t3_hbm, t4_hbm,
                   tab_hbm, out_hbm,
                   tblk, obuf, xv, idxv, wbuf, rows0, rows1, outv,
                   semt, semb, semg0, semg1, semo):
    cid = lax.axis_index("c")
    sid = lax.axis_index("s")
    wid = cid * NS + sid
    bimg = wid // (NWORK // B)
    woff = (wid % (NWORK // B)) * SPW

    iota = lax.iota(jnp.int32, 16)
    semg = (semg0, semg1)
    rowsv = (rows0, rows1)

    for tex, (wl, roff) in zip((t1_hbm, t2_hbm, t3_hbm, t4_hbm), LEVELS):
        lpt = wl // NS
        by = BW // wl
        nblk = lpt // by

        def stage(blk, par, tex=tex, wl=wl, lpt=lpt, by=by):
            base_line = sid * lpt + blk * by
            for li in range(by):
                pltpu.async_copy(tex.at[:, base_line + li, :],
                                 tblk.at[par, :, pl.ds(li * wl, wl)], semt)

        stage(0, 0)

        @pl.loop(0, nblk, step=2)
        def _build(i, tex=tex, wl=wl, roff=roff, lpt=lpt, by=by, nblk=nblk):
            for par in range(2):
                blk = i + par

                @pl.when(blk < nblk)
                def _do(blk=blk, par=par):
                    @pl.when(blk + 1 < nblk)
                    def _prefetch():
                        stage(blk + 1, 1 - par)
                    for li in range(by):
                        pltpu.make_async_copy(
                            tex.at[:, li, :],
                            tblk.at[par, :, pl.ds(li * wl, wl)], semt).wait()
                    @pl.when(blk >= 2)
                    def _drain_w():
                        pltpu.make_async_copy(
                            obuf.at[par], tab_hbm.at[cid, pl.ds(0, BW)],
                            semb).wait()

                    def grp(gi, _):
                        start = gi * 16
                        rowi = start + iota
                        for f in range(FEAT):
                            a = tblk[par, f, pl.ds(start, 16)]
                            bb = tblk[par, f, pl.ds(start + 1, 16)]
                            w = plsc.pack(
                                a, bb, format=plsc.PackFormat.INTERLEAVED)
                            plsc.store_scatter(
                                obuf.at[par],
                                [rowi, jnp.full((16,), f, jnp.int32)],
                                plsc.bitcast(w, jnp.int32))
                        return 0

                    lax.fori_loop(0, BW // 16, grp, 0, unroll=2)
                    rowbase = roff + (sid * lpt + blk * by) * wl
                    pltpu.async_copy(obuf.at[par],
                                     tab_hbm.at[cid, pl.ds(rowbase, BW)],
                                     semb)

        for _ in range(min(2, nblk)):
            pltpu.make_async_copy(
                obuf.at[0], tab_hbm.at[cid, pl.ds(0, BW)], semb).wait()

    plsc.subcore_barrier()
    table_hbm = tab_hbm.at[cid]

    def produce(nxt, sb):
        xrow = nxt & (XBLK - 1)
        for g in range(C // 16):
            sl = pl.ds(g * 16, 16)
            gx = xv[xrow, 0, sl]
            gy = xv[xrow, 1, sl]
            tx = ((gx * 2.0 - 1.0) + 1.0) * 0.5
            ty = ((gy * 2.0 - 1.0) + 1.0) * 0.5
            for l, (wl, roff) in enumerate(LEVELS):
                ix = tx * float(wl - 1)
                iy = ty * float(wl - 1)
                x0 = jnp.minimum(ix.astype(jnp.int32), wl - 2)
                y0 = jnp.minimum(iy.astype(jnp.int32), wl - 2)
                fx1 = ix - x0.astype(jnp.float32)
                fy1 = iy - y0.astype(jnp.float32)
                fx0 = 1.0 - fx1
                fy0 = 1.0 - fy1
                i00 = roff + y0 * wl + x0
                idxv[sb, 2 * l + 0, sl] = i00
                idxv[sb, 2 * l + 1, sl] = i00 + wl
                wbuf[sb, 4 * l + 0, sl] = fy0 * fx0
                wbuf[sb, 4 * l + 1, sl] = fy0 * fx1
                wbuf[sb, 4 * l + 2, sl] = fy1 * fx0
                wbuf[sb, 4 * l + 3, sl] = fy1 * fx1
        for j in range(NJ):
            pltpu.async_copy(table_hbm.at[idxv.at[sb, j]],
                             rowsv[sb].at[pl.ds(j * C, C)], semg[sb])

    def consume(cur, sb):
        rows = rowsv[sb]
        fcv = [jnp.full((16,), f, jnp.int32) for f in range(FEAT)]
        for g in range(C // 16):
            slg = pl.ds(g * 16, 16)

            def jbody(j, acc):
                w0 = wbuf[sb, 2 * j, slg]
                w1 = wbuf[sb, 2 * j + 1, slg]
                rowv = j * C + (g * 16) + iota
                new = []
                for f in range(FEAT):
                    word = plsc.load_gather(rows, [rowv, fcv[f]])
                    bf = plsc.bitcast(word, jnp.bfloat16)
                    v0, v1 = plsc.unpack(
                        bf, format=plsc.PackFormat.INTERLEAVED)
                    new.append(acc[f] + w0 * v0 + w1 * v1)
                return tuple(new)

            acc0 = tuple(jnp.zeros((16,), jnp.float32) for _ in range(FEAT))
            acc = lax.fori_loop(0, NJ, jbody, acc0, unroll=2)
            for f in range(FEAT):
                outv[sb, f, slg] = acc[f]
        pltpu.async_copy(outv.at[sb],
                         out_hbm.at[bimg, :, pl.ds(woff + cur * C, C)],
                         semo)

    pltpu.sync_copy(xq_hbm.at[wid, pl.ds(0, XBLK)], xv)
    produce(0, 0)

    @pl.loop(0, NCHUNK, step=2)
    def _outer(i):
        for b in range(2):
            cur = i + b
            nxt = cur + 1

            @pl.when(nxt < NCHUNK)
            def _stage_and_fire():
                @pl.when((nxt & (XBLK - 1)) == 0)
                def _stage_x():
                    pltpu.sync_copy(xq_hbm.at[wid, pl.ds(nxt, XBLK)], xv)
                produce(nxt, 1 - b)

            pltpu.make_async_copy(table_hbm.at[pl.ds(0, NJ * C)],
                                  rowsv[b], semg[b]).wait()

            @pl.when(cur >= 2)
            def _drain_out():
                pltpu.make_async_copy(
                    outv.at[b], out_hbm.at[0, :, pl.ds(0, C)], semo).wait()

            consume(cur, b)

    for b in range(2):
        pltpu.make_async_copy(
            outv.at[b], out_hbm.at[0, :, pl.ds(0, C)], semo).wait()


def kernel(x, tex1, tex2, tex3, tex4):
    xq = jnp.transpose(
        x.reshape(NWORK, NCHUNK, C, 2), (0, 1, 3, 2))
    _, out = _tex_sc_kernel(xq, tex1, tex2, tex3, tex4)
    return out.reshape(B, FEAT, HG, WG)


# speedup vs baseline: 1.1965x; 1.1965x over previous
import functools

import jax
import jax.numpy as jnp
from jax import lax
from jax.experimental import pallas as pl
from jax.experimental.pallas import tpu as pltpu
from jax.experimental.pallas import tpu_sc as plsc

FEAT = 16
B = 4
HG = 512
WG = 512
N = B * HG * WG
HW = HG * WG

NC = 2
NS = 16
NWORK = NC * NS
SPW = N // NWORK
C = 128
NCHUNK = SPW // C
NL = 4
NJ = 8
NK = 16
XBLK = 8
BW = 1024

LEVELS = ((1024, 0), (512, 1048576), (256, 1310720), (128, 1376256))
VTOT = 1392640


@functools.partial(
    pl.kernel,
    out_type=(
        jax.ShapeDtypeStruct((NC, VTOT, FEAT), jnp.int32),
        jax.ShapeDtypeStruct((B, FEAT, HW), jnp.float32),
    ),
    mesh=plsc.VectorSubcoreMesh(
        core_axis_name="c", subcore_axis_name="s",
        num_cores=NC, num_subcores=NS,
    ),
    compiler_params=pltpu.CompilerParams(
        needs_layout_passes=False, use_tc_tiling_on_sc=False),
    scratch_types=[
        pltpu.VMEM((2, FEAT, BW + 16), jnp.float32),
        pltpu.VMEM((2, BW, FEAT), jnp.int32),
        pltpu.VMEM((XBLK, 2, C), jnp.float32),
        pltpu.VMEM((2, NJ, C), jnp.int32),
        pltpu.VMEM((2, NK, C), jnp.float32),
        pltpu.VMEM((NJ * C, FEAT), jnp.int32),
        pltpu.VMEM((NJ * C, FEAT), jnp.int32),
        pltpu.VMEM((2, FEAT, C), jnp.float32),
        pltpu.SemaphoreType.DMA,
        pltpu.SemaphoreType.DMA,
        pltpu.SemaphoreType.DMA,
        pltpu.SemaphoreType.DMA,
        pltpu.SemaphoreType.DMA,
    ],
)
def _tex_sc_kernel(xq_hbm, t1_hbm, t2_hbm, t3_hbm, t4_hbm,
                   tab_hbm, out_hbm,
                   tblk, obuf, xv, idxv, wbuf, rows0, rows1, outv,
                   semt, semb, semg0, semg1, semo):
    cid = lax.axis_index("c")
    sid = lax.axis_index("s")
    wid = cid * NS + sid
    bimg = wid // (NWORK // B)
    woff = (wid % (NWORK // B)) * SPW

    iota = lax.iota(jnp.int32, 16)
    semg = (semg0, semg1)
    rowsv = (rows0, rows1)

    for tex, (wl, roff) in zip((t1_hbm, t2_hbm, t3_hbm, t4_hbm), LEVELS):
        lpt = wl // NS
        by = BW // wl
        nblk = lpt // by

        def stage(blk, par, tex=tex, wl=wl, lpt=lpt, by=by):
            base_line = sid * lpt + blk * by
            for li in range(by):
                pltpu.async_copy(tex.at[:, base_line + li, :],
                                 tblk.at[par, :, pl.ds(li * wl, wl)], semt)

        stage(0, 0)

        @pl.loop(0, nblk, step=2)
        def _build(i, tex=tex, wl=wl, roff=roff, lpt=lpt, by=by, nblk=nblk):
            for par in range(2):
                blk = i + par

                @pl.when(blk < nblk)
                def _do(blk=blk, par=par):
                    @pl.when(blk + 1 < nblk)
                    def _prefetch():
                        stage(blk + 1, 1 - par)
                    for li in range(by):
                        pltpu.make_async_copy(
                            tex.at[:, li, :],
                            tblk.at[par, :, pl.ds(li * wl, wl)], semt).wait()
                    @pl.when(blk >= 2)
                    def _drain_w():
                        pltpu.make_async_copy(
                            obuf.at[par], tab_hbm.at[cid, pl.ds(0, BW)],
                            semb).wait()

                    def grp(gi, _):
                        start = gi * 16
                        rowi = start + iota
                        for f in range(FEAT):
                            a = tblk[par, f, pl.ds(start, 16)]
                            bb = tblk[par, f, pl.ds(start + 1, 16)]
                            w = plsc.pack(
                                a, bb, format=plsc.PackFormat.INTERLEAVED)
                            plsc.store_scatter(
                                obuf.at[par],
                                [rowi, jnp.full((16,), f, jnp.int32)],
                                plsc.bitcast(w, jnp.int32))
                        return 0

                    lax.fori_loop(0, BW // 16, grp, 0, unroll=2)
                    rowbase = roff + (sid * lpt + blk * by) * wl
                    pltpu.async_copy(obuf.at[par],
                                     tab_hbm.at[cid, pl.ds(rowbase, BW)],
                                     semb)

        for _ in range(min(2, nblk)):
            pltpu.make_async_copy(
                obuf.at[0], tab_hbm.at[cid, pl.ds(0, BW)], semb).wait()

    plsc.subcore_barrier()
    table_hbm = tab_hbm.at[cid]

    def produce(nxt, sb):
        xrow = nxt & (XBLK - 1)
        for g in range(C // 16):
            sl = pl.ds(g * 16, 16)
            gx = xv[xrow, 0, sl]
            gy = xv[xrow, 1, sl]
            tx = ((gx * 2.0 - 1.0) + 1.0) * 0.5
            ty = ((gy * 2.0 - 1.0) + 1.0) * 0.5
            for l, (wl, roff) in enumerate(LEVELS):
                ix = tx * float(wl - 1)
                iy = ty * float(wl - 1)
                x0 = jnp.minimum(ix.astype(jnp.int32), wl - 2)
                y0 = jnp.minimum(iy.astype(jnp.int32), wl - 2)
                fx1 = ix - x0.astype(jnp.float32)
                fy1 = iy - y0.astype(jnp.float32)
                fx0 = 1.0 - fx1
                fy0 = 1.0 - fy1
                i00 = roff + y0 * wl + x0
                idxv[sb, 2 * l + 0, sl] = i00
                idxv[sb, 2 * l + 1, sl] = i00 + wl
                wbuf[sb, 4 * l + 0, sl] = fy0 * fx0
                wbuf[sb, 4 * l + 1, sl] = fy0 * fx1
                wbuf[sb, 4 * l + 2, sl] = fy1 * fx0
                wbuf[sb, 4 * l + 3, sl] = fy1 * fx1
        for j in range(NJ):
            pltpu.async_copy(table_hbm.at[idxv.at[sb, j]],
                             rowsv[sb].at[pl.ds(j * C, C)], semg[sb])

    def consume(cur, sb):
        rows = rowsv[sb]
        fcv = [jnp.full((16,), f, jnp.int32) for f in range(FEAT)]
        for g in range(C // 16):
            slg = pl.ds(g * 16, 16)

            def jbody(j, acc):
                w0 = wbuf[sb, 2 * j, slg]
                w1 = wbuf[sb, 2 * j + 1, slg]
                rowv = j * C + (g * 16) + iota
                new = []
                for f in range(FEAT):
                    word = plsc.load_gather(rows, [rowv, fcv[f]])
                    bf = plsc.bitcast(word, jnp.bfloat16)
                    v0, v1 = plsc.unpack(
                        bf, format=plsc.PackFormat.INTERLEAVED)
                    new.append(acc[f] + w0 * v0 + w1 * v1)
                return tuple(new)

            acc0 = tuple(jnp.zeros((16,), jnp.float32) for _ in range(FEAT))
            acc = lax.fori_loop(0, NJ, jbody, acc0)
            for f in range(FEAT):
                outv[sb, f, slg] = acc[f]
        pltpu.async_copy(outv.at[sb],
                         out_hbm.at[bimg, :, pl.ds(woff + cur * C, C)],
                         semo)

    pltpu.sync_copy(xq_hbm.at[wid, pl.ds(0, XBLK)], xv)
    produce(0, 0)

    @pl.loop(0, NCHUNK, step=2)
    def _outer(i):
        for b in range(2):
            cur = i + b
            nxt = cur + 1

            @pl.when(nxt < NCHUNK)
            def _stage_and_fire():
                @pl.when((nxt & (XBLK - 1)) == 0)
                def _stage_x():
                    pltpu.sync_copy(xq_hbm.at[wid, pl.ds(nxt, XBLK)], xv)
                produce(nxt, 1 - b)

            pltpu.make_async_copy(table_hbm.at[pl.ds(0, NJ * C)],
                                  rowsv[b], semg[b]).wait()

            @pl.when(cur >= 2)
            def _drain_out():
                pltpu.make_async_copy(
                    outv.at[b], out_hbm.at[0, :, pl.ds(0, C)], semo).wait()

            consume(cur, b)

    for b in range(2):
        pltpu.make_async_copy(
            outv.at[b], out_hbm.at[0, :, pl.ds(0, C)], semo).wait()


def kernel(x, tex1, tex2, tex3, tex4):
    xq = jnp.transpose(
        x.reshape(NWORK, NCHUNK, C, 2), (0, 1, 3, 2))
    _, out = _tex_sc_kernel(xq, tex1, tex2, tex3, tex4)
    return out.reshape(B, FEAT, HG, WG)
